# Initial kernel scaffold; baseline (speedup 1.0000x reference)
import jax
import jax.numpy as jnp
from jax.experimental import pallas as pl


def kernel(y_pred, labels, labels_ctrl, mask_full, mask_ctrl, condition_):
    raise NotImplementedError("write your pallas kernel here")


# speedup vs baseline: 1.8728x; 1.8728x over previous
import functools

import jax
import jax.numpy as jnp
from jax.experimental import pallas as pl
from jax.experimental.pallas import tpu as pltpu

EPS = 1e-06

_B, _S = 4096, 2048
_BB = 256
_NB = _B // _BB


def _body(yp_ref, lab_ref, ctl_ref, mf_ref, mc_ref, out_ref, acc_ref):
    i = pl.program_id(0)

    @pl.when(i == 0)
    def _init():
        for k in range(19):
            acc_ref[k] = 0.0

    p0 = yp_ref[:, 0, :]
    p1 = yp_ref[:, 1, :]
    t = lab_ref[...]
    tc = ctl_ref[...]
    mf = mf_ref[...].astype(jnp.float32)
    mc = mc_ref[...].astype(jnp.float32)

    full = p0 + p1
    diff = t - tc
    md = mf * mc

    def sums(p, t_, m, base):
        pm = p * m
        tm = t_ * m
        acc_ref[base + 0] += jnp.sum(m)
        acc_ref[base + 1] += jnp.sum(pm)
        acc_ref[base + 2] += jnp.sum(tm)
        acc_ref[base + 3] += jnp.sum(pm * t_)
        acc_ref[base + 4] += jnp.sum(pm * p)
        acc_ref[base + 5] += jnp.sum(tm * t_)

    sums(p0, tc, mc, 0)
    sums(full, t, mf, 6)
    acc_ref[18] += jnp.sum(jnp.abs(full - t) * mf)
    sums(p1, diff, md, 12)

    @pl.when(i == _NB - 1)
    def _fin():
        def corr(base):
            n = acc_ref[base + 0]
            sp = acc_ref[base + 1]
            st = acc_ref[base + 2]
            spt = acc_ref[base + 3]
            spp = acc_ref[base + 4]
            stt = acc_ref[base + 5]
            dot = spt - sp * st / n
            na = jnp.sqrt(spp - sp * sp / n)
            nb = jnp.sqrt(stt - st * st / n)
            return dot / (jnp.maximum(na, EPS) * jnp.maximum(nb, EPS))

        corr_ctrl = corr(0)
        corr_full = corr(6)
        corr_diff = corr(12)
        l1 = jnp.sqrt(acc_ref[18] / acc_ref[6])
        out_ref[0] = 1.0 - corr_ctrl
        out_ref[1] = (1.0 - corr_full) + l1
        out_ref[2] = corr_full
        out_ref[3] = l1
        out_ref[4] = 1.0 - corr_diff


@functools.partial(jax.jit, static_argnums=())
def _reduce(y_pred, labels, labels_ctrl, mask_full, mask_ctrl):
    return pl.pallas_call(
        _body,
        grid=(_NB,),
        in_specs=[
            pl.BlockSpec((_BB, 2, _S), lambda i: (i, 0, 0)),
            pl.BlockSpec((_BB, _S), lambda i: (i, 0)),
            pl.BlockSpec((_BB, _S), lambda i: (i, 0)),
            pl.BlockSpec((_BB, _S), lambda i: (i, 0)),
            pl.BlockSpec((_BB, _S), lambda i: (i, 0)),
        ],
        out_specs=pl.BlockSpec(memory_space=pltpu.SMEM),
        out_shape=jax.ShapeDtypeStruct((8,), jnp.float32),
        scratch_shapes=[pltpu.SMEM((19,), jnp.float32)],
    )(y_pred, labels, labels_ctrl, mask_full, mask_ctrl)


def kernel(y_pred, labels, labels_ctrl, mask_full, mask_ctrl, condition_):
    out = _reduce(y_pred, labels, labels_ctrl, mask_full, mask_ctrl)
    loss_ctrl, loss_full, perf, l1, loss_depr = (
        out[0], out[1], out[2], out[3], out[4])
    loss = jnp.where(condition_ != 64,
                     loss_ctrl + loss_depr + loss_full,
                     loss_ctrl + loss_full)
    return (loss, perf, l1)


# speedup vs baseline: 2.8687x; 1.5318x over previous
import functools

import jax
import jax.numpy as jnp
from jax.experimental import pallas as pl
from jax.experimental.pallas import tpu as pltpu

EPS = 1e-06

_B, _S = 4096, 2048
_BB = 128
_NB = _B // _BB


def _body(yp_ref, lab_ref, ctl_ref, mf_ref, mc_ref, out_ref, acc_ref):
    i = pl.program_id(0)

    @pl.when(i == 0)
    def _init():
        acc_ref[...] = jnp.zeros_like(acc_ref)

    p0 = yp_ref[:, 0, :]
    p1 = yp_ref[:, 1, :]
    t = lab_ref[...]
    tc = ctl_ref[...]
    mf = mf_ref[...]
    mc = mc_ref[...]
    md = mf & mc

    full = p0 + p1
    diff = t - tc

    def fold(x):
        return jnp.sum(x.reshape(_BB // 8, 8, _S), axis=0)

    def sums(p, t_, m, base):
        u = jnp.where(m, p, 0.0)
        v = jnp.where(m, t_, 0.0)
        one = jnp.where(m, 1.0, 0.0)
        acc_ref[base + 0] += fold(one)
        acc_ref[base + 1] += fold(u)
        acc_ref[base + 2] += fold(v)
        acc_ref[base + 3] += fold(u * v)
        acc_ref[base + 4] += fold(u * u)
        acc_ref[base + 5] += fold(v * v)
        return u, v

    sums(p0, tc, mc, 0)
    u2, v2 = sums(full, t, mf, 6)
    acc_ref[18] += fold(jnp.abs(u2 - v2))
    sums(p1, diff, md, 12)

    @pl.when(i == _NB - 1)
    def _fin():
        def corr(base):
            n = jnp.sum(acc_ref[base + 0])
            sp = jnp.sum(acc_ref[base + 1])
            st = jnp.sum(acc_ref[base + 2])
            spt = jnp.sum(acc_ref[base + 3])
            spp = jnp.sum(acc_ref[base + 4])
            stt = jnp.sum(acc_ref[base + 5])
            dot = spt - sp * st / n
            na = jnp.sqrt(spp - sp * sp / n)
            nb = jnp.sqrt(stt - st * st / n)
            return dot / (jnp.maximum(na, EPS) * jnp.maximum(nb, EPS)), n

        corr_ctrl, _ = corr(0)
        corr_full, n2 = corr(6)
        corr_diff, _ = corr(12)
        l1 = jnp.sqrt(jnp.sum(acc_ref[18]) / n2)
        out_ref[0] = 1.0 - corr_ctrl
        out_ref[1] = (1.0 - corr_full) + l1
        out_ref[2] = corr_full
        out_ref[3] = l1
        out_ref[4] = 1.0 - corr_diff


@jax.jit
def _reduce(y_pred, labels, labels_ctrl, mask_full, mask_ctrl):
    return pl.pallas_call(
        _body,
        grid=(_NB,),
        in_specs=[
            pl.BlockSpec((_BB, 2, _S), lambda i: (i, 0, 0)),
            pl.BlockSpec((_BB, _S), lambda i: (i, 0)),
            pl.BlockSpec((_BB, _S), lambda i: (i, 0)),
            pl.BlockSpec((_BB, _S), lambda i: (i, 0)),
            pl.BlockSpec((_BB, _S), lambda i: (i, 0)),
        ],
        out_specs=pl.BlockSpec(memory_space=pltpu.SMEM),
        out_shape=jax.ShapeDtypeStruct((8,), jnp.float32),
        scratch_shapes=[pltpu.VMEM((19, 8, _S), jnp.float32)],
    )(y_pred, labels, labels_ctrl, mask_full, mask_ctrl)


def kernel(y_pred, labels, labels_ctrl, mask_full, mask_ctrl, condition_):
    out = _reduce(y_pred, labels, labels_ctrl, mask_full, mask_ctrl)
    loss_ctrl, loss_full, perf, l1, loss_depr = (
        out[0], out[1], out[2], out[3], out[4])
    loss = jnp.where(condition_ != 64,
                     loss_ctrl + loss_depr + loss_full,
                     loss_ctrl + loss_full)
    return (loss, perf, l1)
